# baseline (speedup 1.0000x reference)
import jax
import jax.numpy as jnp
from jax.experimental import pallas as pl

_TBLK = 8


def _add_body(emb_ref, x_ref, o_ref):
    o_ref[...] = x_ref[...] + emb_ref[...][:, None, :]


def kernel(x, table):
    B, T, N, C = x.shape
    R = N * C // 128
    x2 = x.reshape(B, T, R, 128)
    table2 = jnp.concatenate([table, table], axis=1)

    out2 = pl.pallas_call(
        _add_body,
        grid=(B, T // _TBLK),
        in_specs=[
            pl.BlockSpec((_TBLK, 128), lambda b, t: (t, 0)),
            pl.BlockSpec((1, _TBLK, R, 128), lambda b, t: (b, t, 0, 0)),
        ],
        out_specs=pl.BlockSpec((1, _TBLK, R, 128), lambda b, t: (b, t, 0, 0)),
        out_shape=jax.ShapeDtypeStruct((B, T, R, 128), x.dtype),
    )(table2, x2)
    return out2.reshape(B, T, N, C)
